# Initial kernel scaffold; baseline (speedup 1.0000x reference)
import functools

import jax
import jax.numpy as jnp
from jax import lax
from jax.experimental import pallas as pl
from jax.experimental.pallas import tpu as pltpu

N = 100000
D = 128
K = 64
BLK = 5000
NBLK = N // BLK


def _body(x_ref, ct_ref, c_ref, out_ref, acc_ref, cnt_ref):
    i = pl.program_id(0)

    @pl.when(i == 0)
    def _init():
        acc_ref[...] = jnp.zeros_like(acc_ref)
        cnt_ref[...] = jnp.zeros_like(cnt_ref)

    x = x_ref[...]
    ct = ct_ref[...]
    c2 = jnp.sum(ct * ct, axis=0, keepdims=True)
    prod = lax.dot_general(
        x, ct, (((1,), (0,)), ((), ())),
        preferred_element_type=jnp.float32,
        precision=lax.Precision.HIGHEST,
    )
    scores = c2 - 2.0 * prod
    iota = lax.broadcasted_iota(jnp.int32, (BLK, K), 1)
    m = jnp.min(scores, axis=1, keepdims=True)
    is_min = scores == m
    first = jnp.min(jnp.where(is_min, iota, K), axis=1, keepdims=True)
    onehot = (iota == first).astype(jnp.float32)

    acc_ref[...] += lax.dot_general(
        onehot, x, (((0,), (0,)), ((), ())),
        preferred_element_type=jnp.float32,
        precision=lax.Precision.HIGHEST,
    )
    ones8 = jnp.ones((BLK, 8), dtype=jnp.float32)
    cnt_ref[...] += lax.dot_general(
        onehot, ones8, (((0,), (0,)), ((), ())),
        preferred_element_type=jnp.float32,
        precision=lax.Precision.HIGHEST,
    )

    @pl.when(i == NBLK - 1)
    def _finalize():
        cnt = jnp.broadcast_to(cnt_ref[...][:, :1], (K, D))
        means = acc_ref[...] / jnp.maximum(cnt, 1.0)
        out_ref[...] = jnp.where(cnt > 0.0, means, c_ref[...])


@functools.partial(jax.jit)
def kernel(features, cluster_centers):
    centers_t = cluster_centers.T
    return pl.pallas_call(
        _body,
        grid=(NBLK,),
        in_specs=[
            pl.BlockSpec((BLK, D), lambda i: (i, 0)),
            pl.BlockSpec((D, K), lambda i: (0, 0)),
            pl.BlockSpec((K, D), lambda i: (0, 0)),
        ],
        out_specs=pl.BlockSpec((K, D), lambda i: (0, 0)),
        out_shape=jax.ShapeDtypeStruct((K, D), jnp.float32),
        scratch_shapes=[
            pltpu.VMEM((K, D), jnp.float32),
            pltpu.VMEM((K, 8), jnp.float32),
        ],
    )(features, centers_t, cluster_centers)


# speedup vs baseline: 3.7047x; 3.7047x over previous
import functools

import jax
import jax.numpy as jnp
from jax import lax
from jax.experimental import pallas as pl
from jax.experimental.pallas import tpu as pltpu

N = 100000
D = 128
K = 64
BLK = 5000
NBLK = N // BLK


def _body(x_ref, ct_ref, c_ref, out_ref, acc_ref, cnt_ref):
    i = pl.program_id(0)

    @pl.when(i == 0)
    def _init():
        acc_ref[...] = jnp.zeros_like(acc_ref)
        cnt_ref[...] = jnp.zeros_like(cnt_ref)

    x = x_ref[...]
    ct = ct_ref[...]
    f2 = jnp.sum(x * x, axis=1, keepdims=True)
    c2 = jnp.sum(ct * ct, axis=0, keepdims=True)
    prod = lax.dot_general(
        x, ct, (((1,), (0,)), ((), ())),
        preferred_element_type=jnp.float32,
    )
    sq = f2 + c2 - 2.0 * prod
    scores = jnp.sqrt(jnp.maximum(sq, 0.0))
    iota = lax.broadcasted_iota(jnp.int32, (BLK, K), 1)
    m = jnp.min(scores, axis=1, keepdims=True)
    is_min = scores == m
    first = jnp.min(jnp.where(is_min, iota, K), axis=1, keepdims=True)
    onehot = (iota == first).astype(jnp.float32)

    acc_ref[...] += lax.dot_general(
        onehot, x, (((0,), (0,)), ((), ())),
        preferred_element_type=jnp.float32,
        precision=lax.Precision.HIGHEST,
    )
    ones8 = jnp.ones((BLK, 8), dtype=jnp.float32)
    cnt_ref[...] += lax.dot_general(
        onehot, ones8, (((0,), (0,)), ((), ())),
        preferred_element_type=jnp.float32,
        precision=lax.Precision.HIGHEST,
    )

    @pl.when(i == NBLK - 1)
    def _finalize():
        cnt = jnp.broadcast_to(cnt_ref[...][:, :1], (K, D))
        means = acc_ref[...] / jnp.maximum(cnt, 1.0)
        out_ref[...] = jnp.where(cnt > 0.0, means, c_ref[...])


@functools.partial(jax.jit)
def kernel(features, cluster_centers):
    centers_t = cluster_centers.T
    return pl.pallas_call(
        _body,
        grid=(NBLK,),
        in_specs=[
            pl.BlockSpec((BLK, D), lambda i: (i, 0)),
            pl.BlockSpec((D, K), lambda i: (0, 0)),
            pl.BlockSpec((K, D), lambda i: (0, 0)),
        ],
        out_specs=pl.BlockSpec((K, D), lambda i: (0, 0)),
        out_shape=jax.ShapeDtypeStruct((K, D), jnp.float32),
        scratch_shapes=[
            pltpu.VMEM((K, D), jnp.float32),
            pltpu.VMEM((K, 8), jnp.float32),
        ],
    )(features, centers_t, cluster_centers)


# speedup vs baseline: 6.0030x; 1.6204x over previous
import functools

import jax
import jax.numpy as jnp
from jax import lax
from jax.experimental import pallas as pl
from jax.experimental.pallas import tpu as pltpu

N = 100000
D = 128
K = 64
BLK = 5000
NBLK = N // BLK


def _body(x_ref, ct_ref, c_ref, out_ref, acc_ref, cnt_ref):
    i = pl.program_id(0)

    @pl.when(i == 0)
    def _init():
        acc_ref[...] = jnp.zeros_like(acc_ref)
        cnt_ref[...] = jnp.zeros_like(cnt_ref)

    x = x_ref[...]
    ct = ct_ref[...]
    f2 = jnp.sum(x * x, axis=1, keepdims=True)
    c2 = jnp.sum(ct * ct, axis=0, keepdims=True)
    prod = lax.dot_general(
        x, ct, (((1,), (0,)), ((), ())),
        preferred_element_type=jnp.float32,
    )
    sq = f2 + c2 - 2.0 * prod
    scores = jnp.sqrt(jnp.maximum(sq, 0.0))
    iota = lax.broadcasted_iota(jnp.int32, (BLK, K), 1)
    m = jnp.min(scores, axis=1, keepdims=True)
    is_min = scores == m
    first = jnp.min(jnp.where(is_min, iota, K), axis=1, keepdims=True)
    onehot = jnp.where(iota == first, 1.0, 0.0)

    acc_ref[...] += lax.dot_general(
        onehot, x, (((0,), (0,)), ((), ())),
        preferred_element_type=jnp.float32,
    )
    ones8 = jnp.ones((BLK, 8), dtype=jnp.float32)
    cnt_ref[...] += lax.dot_general(
        onehot, ones8, (((0,), (0,)), ((), ())),
        preferred_element_type=jnp.float32,
    )

    @pl.when(i == NBLK - 1)
    def _finalize():
        cnt = jnp.broadcast_to(cnt_ref[...][:, :1], (K, D))
        means = acc_ref[...] / jnp.maximum(cnt, 1.0)
        out_ref[...] = jnp.where(cnt > 0.0, means, c_ref[...])


@functools.partial(jax.jit)
def kernel(features, cluster_centers):
    centers_t = cluster_centers.T
    return pl.pallas_call(
        _body,
        grid=(NBLK,),
        in_specs=[
            pl.BlockSpec((BLK, D), lambda i: (i, 0)),
            pl.BlockSpec((D, K), lambda i: (0, 0)),
            pl.BlockSpec((K, D), lambda i: (0, 0)),
        ],
        out_specs=pl.BlockSpec((K, D), lambda i: (0, 0)),
        out_shape=jax.ShapeDtypeStruct((K, D), jnp.float32),
        scratch_shapes=[
            pltpu.VMEM((K, D), jnp.float32),
            pltpu.VMEM((K, 8), jnp.float32),
        ],
    )(features, centers_t, cluster_centers)


# speedup vs baseline: 11.1164x; 1.8518x over previous
import functools

import jax
import jax.numpy as jnp
from jax import lax
from jax.experimental import pallas as pl
from jax.experimental.pallas import tpu as pltpu

N = 100000
D = 128
K = 64
BLK = 5000
NBLK = N // BLK


def _body(x_ref, ct_ref, c_ref, out_ref, acc_ref, cnt_ref):
    i = pl.program_id(0)

    @pl.when(i == 0)
    def _init():
        acc_ref[...] = jnp.zeros_like(acc_ref)
        cnt_ref[...] = jnp.zeros_like(cnt_ref)

    x = x_ref[...]
    ct = ct_ref[...]
    c2 = 0.25 * jnp.sum(ct * ct, axis=0, keepdims=True)
    scores = c2 + lax.dot_general(
        x, ct, (((1,), (0,)), ((), ())),
        preferred_element_type=jnp.float32,
    )
    m = jnp.min(scores, axis=1, keepdims=True)
    onehot = jnp.where(scores == m, 1.0, 0.0)

    acc_ref[...] += lax.dot_general(
        onehot, x, (((0,), (0,)), ((), ())),
        preferred_element_type=jnp.float32,
    )
    ones8 = jnp.ones((BLK, 8), dtype=jnp.float32)
    cnt_ref[...] += lax.dot_general(
        onehot, ones8, (((0,), (0,)), ((), ())),
        preferred_element_type=jnp.float32,
    )

    @pl.when(i == NBLK - 1)
    def _finalize():
        cnt = jnp.broadcast_to(cnt_ref[...][:, :1], (K, D))
        means = acc_ref[...] / jnp.maximum(cnt, 1.0)
        out_ref[...] = jnp.where(cnt > 0.0, means, c_ref[...])


@functools.partial(jax.jit)
def kernel(features, cluster_centers):
    centers_t = -2.0 * cluster_centers.T
    return pl.pallas_call(
        _body,
        grid=(NBLK,),
        in_specs=[
            pl.BlockSpec((BLK, D), lambda i: (i, 0)),
            pl.BlockSpec((D, K), lambda i: (0, 0)),
            pl.BlockSpec((K, D), lambda i: (0, 0)),
        ],
        out_specs=pl.BlockSpec((K, D), lambda i: (0, 0)),
        out_shape=jax.ShapeDtypeStruct((K, D), jnp.float32),
        scratch_shapes=[
            pltpu.VMEM((K, D), jnp.float32),
            pltpu.VMEM((K, 8), jnp.float32),
        ],
    )(features, centers_t, cluster_centers)


# speedup vs baseline: 17.1245x; 1.5405x over previous
import functools

import jax
import jax.numpy as jnp
from jax import lax
from jax.experimental import pallas as pl
from jax.experimental.pallas import tpu as pltpu

N = 100000
D = 128
K = 64
BLK = 10000
NBLK = N // BLK


def _body(x_ref, ct_ref, c_ref, out_ref, acc_ref, cnt_ref, c2_ref):
    i = pl.program_id(0)

    @pl.when(i == 0)
    def _init():
        acc_ref[...] = jnp.zeros_like(acc_ref)
        cnt_ref[...] = jnp.zeros_like(cnt_ref)
        ct0 = ct_ref[...]
        c2_ref[...] = 0.25 * jnp.sum(ct0 * ct0, axis=0, keepdims=True)

    x = x_ref[...]
    ct = ct_ref[...]
    scores = c2_ref[...] + lax.dot_general(
        x, ct, (((1,), (0,)), ((), ())),
        preferred_element_type=jnp.float32,
    )
    m = jnp.min(scores, axis=1, keepdims=True)
    onehot = jnp.where(scores == m, 1.0, 0.0)

    acc_ref[...] += lax.dot_general(
        onehot, x, (((0,), (0,)), ((), ())),
        preferred_element_type=jnp.float32,
    )
    cnt_ref[...] += jnp.sum(onehot, axis=0, keepdims=True)

    @pl.when(i == NBLK - 1)
    def _finalize():
        riota = lax.broadcasted_iota(jnp.int32, (K, K), 0)
        ciota = lax.broadcasted_iota(jnp.int32, (K, K), 1)
        eye = jnp.where(riota == ciota, 1.0, 0.0)
        cnt_col = lax.dot_general(
            eye, cnt_ref[...], (((1,), (1,)), ((), ())),
            preferred_element_type=jnp.float32,
        )
        cnt = jnp.broadcast_to(cnt_col, (K, D))
        means = acc_ref[...] / jnp.maximum(cnt, 1.0)
        out_ref[...] = jnp.where(cnt > 0.0, means, c_ref[...])


@functools.partial(jax.jit)
def kernel(features, cluster_centers):
    centers_t = -2.0 * cluster_centers.T
    return pl.pallas_call(
        _body,
        grid=(NBLK,),
        in_specs=[
            pl.BlockSpec((BLK, D), lambda i: (i, 0)),
            pl.BlockSpec((D, K), lambda i: (0, 0)),
            pl.BlockSpec((K, D), lambda i: (0, 0)),
        ],
        out_specs=pl.BlockSpec((K, D), lambda i: (0, 0)),
        out_shape=jax.ShapeDtypeStruct((K, D), jnp.float32),
        scratch_shapes=[
            pltpu.VMEM((K, D), jnp.float32),
            pltpu.VMEM((1, K), jnp.float32),
            pltpu.VMEM((1, K), jnp.float32),
        ],
    )(features, centers_t, cluster_centers)
